# initial kernel scaffold (unmeasured)
import jax
import jax.numpy as jnp
from jax import lax
from jax.experimental import pallas as pl
from jax.experimental.pallas import tpu as pltpu

N_DEV = 32
B = 2
SQ = 128
HQ = 4
DH = 64
DM = 512
DQ = HQ * DH
BLK = 64
SCALE = 0.125


def kernel(x, Wq, K_ext, V_ext, Wo):
    def body(x_ref, wq_ref, k_ref, v_ref, wo_ref, out_ref,
             kvt_ref, comm_ref, acc_ref, l_ref, send_sems, recv_sems):
        d = lax.axis_index("i")

        kvt_ref[0] = k_ref[:].transpose(0, 2, 1, 3)
        kvt_ref[1] = v_ref[:].transpose(0, 2, 1, 3)

        for k in range(1, N_DEV):
            @pl.when(d + k <= N_DEV - 1)
            def _(k=k):
                pltpu.make_async_remote_copy(
                    src_ref=kvt_ref,
                    dst_ref=comm_ref.at[k - 1],
                    send_sem=send_sems.at[k - 1],
                    recv_sem=recv_sems.at[k - 1],
                    device_id=(d + k,),
                    device_id_type=pl.DeviceIdType.MESH,
                ).start()

        q = jnp.dot(x_ref[:].reshape(B * SQ, DM), wq_ref[:],
                    preferred_element_type=jnp.float32) * SCALE
        q = q.reshape(B, SQ, HQ, DH).transpose(0, 2, 1, 3)

        def scores_ctx(kt, vt, w_mask=None):
            s = lax.dot_general(q, kt, (((3,), (3,)), ((0, 1), (0, 1))),
                                preferred_element_type=jnp.float32)
            w = jnp.exp(s)
            if w_mask is not None:
                w = jnp.where(w_mask, w, 0.0)
            ctx = lax.dot_general(w, vt, (((3,), (2,)), ((0, 1), (0, 1))),
                                  preferred_element_type=jnp.float32)
            return w, ctx

        qb = lax.broadcasted_iota(jnp.int32, (SQ, SQ), 0) // BLK
        kb = lax.broadcasted_iota(jnp.int32, (SQ, SQ), 1) // BLK
        mask = (kb <= qb)[None, None]
        w, ctx = scores_ctx(kvt_ref[0], kvt_ref[1], mask)
        acc_ref[:] = ctx
        l_ref[:] = jnp.sum(w, axis=3)

        for k in range(1, N_DEV):
            @pl.when(d >= k)
            def _(k=k):
                pltpu.make_async_remote_copy(
                    src_ref=kvt_ref,
                    dst_ref=comm_ref.at[k - 1],
                    send_sem=send_sems.at[k - 1],
                    recv_sem=recv_sems.at[k - 1],
                    device_id=(d,),
                    device_id_type=pl.DeviceIdType.MESH,
                ).wait_recv()
                w, ctx = scores_ctx(comm_ref[k - 1, 0], comm_ref[k - 1, 1])
                acc_ref[:] += ctx
                l_ref[:] += jnp.sum(w, axis=3)

        ctx = acc_ref[:] / l_ref[:][..., None]
        ctx = ctx.transpose(0, 2, 1, 3).reshape(B * SQ, DQ)
        out = jnp.dot(ctx, wo_ref[:], preferred_element_type=jnp.float32)
        out_ref[:] = out.reshape(B, SQ, DM)

        for k in range(1, N_DEV):
            @pl.when(d + k <= N_DEV - 1)
            def _(k=k):
                pltpu.make_async_remote_copy(
                    src_ref=kvt_ref,
                    dst_ref=comm_ref.at[k - 1],
                    send_sem=send_sems.at[k - 1],
                    recv_sem=recv_sems.at[k - 1],
                    device_id=(d,),
                    device_id_type=pl.DeviceIdType.MESH,
                ).wait_send()

    return pl.pallas_call(
        body,
        out_shape=jax.ShapeDtypeStruct((B, SQ, DM), jnp.float32),
        in_specs=[pl.BlockSpec(memory_space=pltpu.VMEM)] * 5,
        out_specs=pl.BlockSpec(memory_space=pltpu.VMEM),
        scratch_shapes=[
            pltpu.VMEM((2, B, HQ, SQ, DH), jnp.float32),
            pltpu.VMEM((N_DEV - 1, 2, B, HQ, SQ, DH), jnp.float32),
            pltpu.VMEM((B, HQ, SQ, DH), jnp.float32),
            pltpu.VMEM((B, HQ, SQ), jnp.float32),
            pltpu.SemaphoreType.DMA((N_DEV - 1,)),
            pltpu.SemaphoreType.DMA((N_DEV - 1,)),
        ],
        compiler_params=pltpu.CompilerParams(collective_id=0),
    )(x, Wq, K_ext, V_ext, Wo)


# baseline (device time: 458721 ns/iter reference)
import jax
import jax.numpy as jnp
from jax import lax
from jax.experimental import pallas as pl
from jax.experimental.pallas import tpu as pltpu

N_DEV = 32
B = 2
SQ = 128
HQ = 4
DH = 64
DM = 512
DQ = HQ * DH
BLK = 64
SCALE = 0.125


def kernel(x, Wq, K_ext, V_ext, Wo):
    def body(x_ref, wq_ref, k_ref, v_ref, wo_ref, out_ref,
             kvt_ref, comm_ref, acc_ref, l_ref, send_sems, recv_sems):
        d = lax.axis_index("i")

        barrier_sem = pltpu.get_barrier_semaphore()
        for t in range(N_DEV):
            @pl.when(d != t)
            def _(t=t):
                pl.semaphore_signal(
                    barrier_sem, inc=1,
                    device_id=(t,), device_id_type=pl.DeviceIdType.MESH,
                )
        pl.semaphore_wait(barrier_sem, N_DEV - 1)

        kvt_ref[0] = k_ref[:].transpose(0, 2, 1, 3).reshape(B * HQ, SQ, DH)
        kvt_ref[1] = v_ref[:].transpose(0, 2, 1, 3).reshape(B * HQ, SQ, DH)

        for k in range(1, N_DEV):
            @pl.when(d + k <= N_DEV - 1)
            def _(k=k):
                pltpu.make_async_remote_copy(
                    src_ref=kvt_ref,
                    dst_ref=comm_ref.at[k - 1],
                    send_sem=send_sems.at[k - 1],
                    recv_sem=recv_sems.at[k - 1],
                    device_id=(d + k,),
                    device_id_type=pl.DeviceIdType.MESH,
                ).start()

        q = jnp.dot(x_ref[:].reshape(B * SQ, DM), wq_ref[:],
                    preferred_element_type=jnp.float32) * SCALE
        q = q.reshape(B, SQ, HQ, DH).transpose(0, 2, 1, 3).reshape(B * HQ, SQ, DH)

        def scores_ctx(kt, vt, w_mask=None):
            s = lax.dot_general(q, kt, (((2,), (2,)), ((0,), (0,))),
                                preferred_element_type=jnp.float32)
            w = jnp.exp(s)
            if w_mask is not None:
                w = jnp.where(w_mask, w, 0.0)
            ctx = lax.dot_general(w, vt, (((2,), (1,)), ((0,), (0,))),
                                  preferred_element_type=jnp.float32)
            return w, ctx

        qb = lax.broadcasted_iota(jnp.int32, (SQ, SQ), 0) // BLK
        kb = lax.broadcasted_iota(jnp.int32, (SQ, SQ), 1) // BLK
        mask = (kb <= qb)[None]
        w, ctx = scores_ctx(kvt_ref[0], kvt_ref[1], mask)
        acc_ref[:] = ctx
        l_ref[:] = jnp.sum(w, axis=2)

        for k in range(1, N_DEV):
            @pl.when(d >= k)
            def _(k=k):
                pltpu.make_async_remote_copy(
                    src_ref=kvt_ref,
                    dst_ref=comm_ref.at[k - 1],
                    send_sem=send_sems.at[k - 1],
                    recv_sem=recv_sems.at[k - 1],
                    device_id=(d,),
                    device_id_type=pl.DeviceIdType.MESH,
                ).wait_recv()
                w, ctx = scores_ctx(comm_ref[k - 1, 0], comm_ref[k - 1, 1])
                acc_ref[:] += ctx
                l_ref[:] += jnp.sum(w, axis=2)

        ctx = acc_ref[:] / l_ref[:][..., None]
        ctx = ctx.reshape(B, HQ, SQ, DH).transpose(0, 2, 1, 3).reshape(B * SQ, DQ)
        out = jnp.dot(ctx, wo_ref[:], preferred_element_type=jnp.float32)
        out_ref[:] = out.reshape(B, SQ, DM)

        for k in range(1, N_DEV):
            @pl.when(d + k <= N_DEV - 1)
            def _(k=k):
                pltpu.make_async_remote_copy(
                    src_ref=kvt_ref,
                    dst_ref=comm_ref.at[k - 1],
                    send_sem=send_sems.at[k - 1],
                    recv_sem=recv_sems.at[k - 1],
                    device_id=(d,),
                    device_id_type=pl.DeviceIdType.MESH,
                ).wait_send()

    return pl.pallas_call(
        body,
        out_shape=jax.ShapeDtypeStruct((B, SQ, DM), jnp.float32),
        in_specs=[pl.BlockSpec(memory_space=pltpu.VMEM)] * 5,
        out_specs=pl.BlockSpec(memory_space=pltpu.VMEM),
        scratch_shapes=[
            pltpu.VMEM((2, B * HQ, SQ, DH), jnp.float32),
            pltpu.VMEM((N_DEV - 1, 2, B * HQ, SQ, DH), jnp.float32),
            pltpu.VMEM((B * HQ, SQ, DH), jnp.float32),
            pltpu.VMEM((B * HQ, SQ), jnp.float32),
            pltpu.SemaphoreType.DMA((N_DEV - 1,)),
            pltpu.SemaphoreType.DMA((N_DEV - 1,)),
        ],
        compiler_params=pltpu.CompilerParams(
            vmem_limit_bytes=60 * 1024 * 1024,
            collective_id=0,
        ),
    )(x, Wq, K_ext, V_ext, Wo)


# device time: 235662 ns/iter; 1.9465x vs baseline; 1.9465x over previous
import jax
import jax.numpy as jnp
from jax import lax
from jax.experimental import pallas as pl
from jax.experimental.pallas import tpu as pltpu

N_DEV = 32
B = 2
SQ = 128
HQ = 4
DH = 64
DM = 512
DQ = HQ * DH
BLK = 64
SCALE = 0.125


def kernel(x, Wq, K_ext, V_ext, Wo):
    def body(x_ref, wq_ref, k_ref, v_ref, wo_ref, out_ref,
             kvt_ref, comm_ref, acc_ref, l_ref, send_sems, recv_sems):
        d = lax.axis_index("i")

        barrier_sem = pltpu.get_barrier_semaphore()
        for t in range(N_DEV):
            @pl.when(d != t)
            def _(t=t):
                pl.semaphore_signal(
                    barrier_sem, inc=1,
                    device_id=(t,), device_id_type=pl.DeviceIdType.MESH,
                )
        pl.semaphore_wait(barrier_sem, N_DEV - 1)

        kvt_ref[0] = k_ref[:].transpose(0, 2, 1, 3).reshape(B * HQ, SQ, DH).astype(jnp.bfloat16)
        kvt_ref[1] = v_ref[:].transpose(0, 2, 1, 3).reshape(B * HQ, SQ, DH).astype(jnp.bfloat16)

        for k in range(1, N_DEV):
            @pl.when(d + k <= N_DEV - 1)
            def _(k=k):
                pltpu.make_async_remote_copy(
                    src_ref=kvt_ref,
                    dst_ref=comm_ref.at[k - 1],
                    send_sem=send_sems.at[k - 1],
                    recv_sem=recv_sems.at[k - 1],
                    device_id=(d + k,),
                    device_id_type=pl.DeviceIdType.MESH,
                ).start()

        q = jnp.dot(x_ref[:].reshape(B * SQ, DM), wq_ref[:],
                    preferred_element_type=jnp.float32) * SCALE
        q = q.reshape(B, SQ, HQ, DH).transpose(0, 2, 1, 3).reshape(B * HQ, SQ, DH)
        q = q.astype(jnp.bfloat16)

        def scores_ctx(kt, vt, w_mask=None):
            s = lax.dot_general(q, kt, (((2,), (2,)), ((0,), (0,))),
                                preferred_element_type=jnp.float32)
            w = jnp.exp(s)
            if w_mask is not None:
                w = jnp.where(w_mask, w, 0.0)
            ctx = lax.dot_general(w.astype(jnp.bfloat16), vt,
                                  (((2,), (1,)), ((0,), (0,))),
                                  preferred_element_type=jnp.float32)
            return w, ctx

        qb = lax.broadcasted_iota(jnp.int32, (SQ, SQ), 0) // BLK
        kb = lax.broadcasted_iota(jnp.int32, (SQ, SQ), 1) // BLK
        mask = (kb <= qb)[None]
        w, ctx = scores_ctx(kvt_ref[0], kvt_ref[1], mask)
        acc_ref[:] = ctx
        l_ref[:] = jnp.sum(w, axis=2)

        for k in range(1, N_DEV):
            @pl.when(d >= k)
            def _(k=k):
                pltpu.make_async_remote_copy(
                    src_ref=kvt_ref,
                    dst_ref=comm_ref.at[k - 1],
                    send_sem=send_sems.at[k - 1],
                    recv_sem=recv_sems.at[k - 1],
                    device_id=(d,),
                    device_id_type=pl.DeviceIdType.MESH,
                ).wait_recv()
                w, ctx = scores_ctx(comm_ref[k - 1, 0], comm_ref[k - 1, 1])
                acc_ref[:] += ctx
                l_ref[:] += jnp.sum(w, axis=2)

        ctx = acc_ref[:] / l_ref[:][..., None]
        ctx = ctx.reshape(B, HQ, SQ, DH).transpose(0, 2, 1, 3).reshape(B * SQ, DQ)
        out = jnp.dot(ctx, wo_ref[:], preferred_element_type=jnp.float32)
        out_ref[:] = out.reshape(B, SQ, DM)

        for k in range(1, N_DEV):
            @pl.when(d + k <= N_DEV - 1)
            def _(k=k):
                pltpu.make_async_remote_copy(
                    src_ref=kvt_ref,
                    dst_ref=comm_ref.at[k - 1],
                    send_sem=send_sems.at[k - 1],
                    recv_sem=recv_sems.at[k - 1],
                    device_id=(d,),
                    device_id_type=pl.DeviceIdType.MESH,
                ).wait_send()

    return pl.pallas_call(
        body,
        out_shape=jax.ShapeDtypeStruct((B, SQ, DM), jnp.float32),
        in_specs=[pl.BlockSpec(memory_space=pltpu.VMEM)] * 5,
        out_specs=pl.BlockSpec(memory_space=pltpu.VMEM),
        scratch_shapes=[
            pltpu.VMEM((2, B * HQ, SQ, DH), jnp.bfloat16),
            pltpu.VMEM((N_DEV - 1, 2, B * HQ, SQ, DH), jnp.bfloat16),
            pltpu.VMEM((B * HQ, SQ, DH), jnp.float32),
            pltpu.VMEM((B * HQ, SQ), jnp.float32),
            pltpu.SemaphoreType.DMA((N_DEV - 1,)),
            pltpu.SemaphoreType.DMA((N_DEV - 1,)),
        ],
        compiler_params=pltpu.CompilerParams(
            vmem_limit_bytes=60 * 1024 * 1024,
            collective_id=0,
        ),
    )(x, Wq, K_ext, V_ext, Wo)


# device time: 43172 ns/iter; 10.6254x vs baseline; 5.4587x over previous
import jax
import jax.numpy as jnp
from jax import lax
from jax.experimental import pallas as pl
from jax.experimental.pallas import tpu as pltpu

N_DEV = 32
B = 2
SQ = 128
HQ = 4
DH = 64
DM = 512
DQ = HQ * DH
BLK = 64
SCALE = 0.125


def kernel(x, Wq, K_ext, V_ext, Wo):
    def body(x_ref, wq_ref, k_ref, v_ref, wo_ref, out_ref,
             kvt_ref, comm_ref, acc_ref, l_ref, send_sems, recv_sems):
        d = lax.axis_index("i")

        barrier_sem = pltpu.get_barrier_semaphore()
        for t in range(N_DEV):
            @pl.when(d != t)
            def _(t=t):
                pl.semaphore_signal(
                    barrier_sem, inc=1,
                    device_id=(t,), device_id_type=pl.DeviceIdType.MESH,
                )
        pl.semaphore_wait(barrier_sem, N_DEV - 1)

        kvt_ref[0] = k_ref[:].transpose(0, 2, 1, 3).reshape(B * HQ, SQ, DH).astype(jnp.bfloat16)
        kvt_ref[1] = v_ref[:].transpose(0, 2, 1, 3).reshape(B * HQ, SQ, DH).astype(jnp.bfloat16)

        @pl.when(d <= N_DEV - 2)
        def _():
            pltpu.make_async_remote_copy(
                src_ref=kvt_ref,
                dst_ref=comm_ref.at[0],
                send_sem=send_sems.at[0],
                recv_sem=recv_sems.at[0],
                device_id=(d + 1,),
                device_id_type=pl.DeviceIdType.MESH,
            ).start()

        q = jnp.dot(x_ref[:].reshape(B * SQ, DM), wq_ref[:],
                    preferred_element_type=jnp.float32) * SCALE
        q = q.reshape(B, SQ, HQ, DH).transpose(0, 2, 1, 3).reshape(B * HQ, SQ, DH)
        q = q.astype(jnp.bfloat16)

        def scores_ctx(kt, vt, w_mask=None):
            s = lax.dot_general(q, kt, (((2,), (2,)), ((0,), (0,))),
                                preferred_element_type=jnp.float32)
            w = jnp.exp(s)
            if w_mask is not None:
                w = jnp.where(w_mask, w, 0.0)
            ctx = lax.dot_general(w.astype(jnp.bfloat16), vt,
                                  (((2,), (1,)), ((0,), (0,))),
                                  preferred_element_type=jnp.float32)
            return w, ctx

        qb = lax.broadcasted_iota(jnp.int32, (SQ, SQ), 0) // BLK
        kb = lax.broadcasted_iota(jnp.int32, (SQ, SQ), 1) // BLK
        mask = (kb <= qb)[None]
        w, ctx = scores_ctx(kvt_ref[0], kvt_ref[1], mask)
        acc_ref[:] = ctx
        l_ref[:] = jnp.sum(w, axis=2)

        for h in range(N_DEV - 1):
            @pl.when(d >= h + 1)
            def _(h=h):
                pltpu.make_async_remote_copy(
                    src_ref=kvt_ref,
                    dst_ref=comm_ref.at[h],
                    send_sem=send_sems.at[h],
                    recv_sem=recv_sems.at[h],
                    device_id=(d,),
                    device_id_type=pl.DeviceIdType.MESH,
                ).wait_recv()
            if h < N_DEV - 2:
                @pl.when((d >= h + 1) & (d <= N_DEV - 2))
                def _(h=h):
                    pltpu.make_async_remote_copy(
                        src_ref=comm_ref.at[h],
                        dst_ref=comm_ref.at[h + 1],
                        send_sem=send_sems.at[h + 1],
                        recv_sem=recv_sems.at[h + 1],
                        device_id=(d + 1,),
                        device_id_type=pl.DeviceIdType.MESH,
                    ).start()
            @pl.when(d >= h + 1)
            def _(h=h):
                w, ctx = scores_ctx(comm_ref[h, 0], comm_ref[h, 1])
                acc_ref[:] += ctx
                l_ref[:] += jnp.sum(w, axis=2)

        ctx = acc_ref[:] / l_ref[:][..., None]
        ctx = ctx.reshape(B, HQ, SQ, DH).transpose(0, 2, 1, 3).reshape(B * SQ, DQ)
        out = jnp.dot(ctx, wo_ref[:], preferred_element_type=jnp.float32)
        out_ref[:] = out.reshape(B, SQ, DM)

        for h in range(N_DEV - 1):
            @pl.when((d >= h) & (d <= N_DEV - 2))
            def _(h=h):
                pltpu.make_async_remote_copy(
                    src_ref=kvt_ref,
                    dst_ref=comm_ref.at[h],
                    send_sem=send_sems.at[h],
                    recv_sem=recv_sems.at[h],
                    device_id=(d,),
                    device_id_type=pl.DeviceIdType.MESH,
                ).wait_send()

    return pl.pallas_call(
        body,
        out_shape=jax.ShapeDtypeStruct((B, SQ, DM), jnp.float32),
        in_specs=[pl.BlockSpec(memory_space=pltpu.VMEM)] * 5,
        out_specs=pl.BlockSpec(memory_space=pltpu.VMEM),
        scratch_shapes=[
            pltpu.VMEM((2, B * HQ, SQ, DH), jnp.bfloat16),
            pltpu.VMEM((N_DEV - 1, 2, B * HQ, SQ, DH), jnp.bfloat16),
            pltpu.VMEM((B * HQ, SQ, DH), jnp.float32),
            pltpu.VMEM((B * HQ, SQ), jnp.float32),
            pltpu.SemaphoreType.DMA((N_DEV - 1,)),
            pltpu.SemaphoreType.DMA((N_DEV - 1,)),
        ],
        compiler_params=pltpu.CompilerParams(
            vmem_limit_bytes=60 * 1024 * 1024,
            collective_id=0,
        ),
    )(x, Wq, K_ext, V_ext, Wo)
